# baseline (speedup 1.0000x reference)
import functools

import jax
import jax.numpy as jnp
from jax import lax
from jax.experimental import pallas as pl
from jax.experimental.pallas import tpu as pltpu
from jax.experimental.pallas import tpu_sc as plsc

_INFO = plsc.get_sparse_core_info()
_NC = _INFO.num_cores
_NS = _INFO.num_subcores
_NW = _NC * _NS
_BW = 1024
_CH = 768


@functools.partial(jax.jit, static_argnames=("vocab", "dim"))
def _sc_detile(tab_t, tail, vocab, dim):
    vmain = (vocab // 128) * 128
    nchunks = vmain // _CH
    npairs = -(-nchunks // (2 * _NW))
    ntail = (vocab - vmain) * dim
    mesh = plsc.VectorSubcoreMesh(core_axis_name="c", subcore_axis_name="s")

    @functools.partial(
        pl.kernel,
        mesh=mesh,
        compiler_params=pltpu.CompilerParams(needs_layout_passes=False),
        out_type=jax.ShapeDtypeStruct((vocab * dim,), jnp.float32),
        scratch_types=[
            pltpu.VMEM((dim, _CH + 1), jnp.float32),
            pltpu.VMEM((dim, _CH + 1), jnp.float32),
            pltpu.VMEM((_CH * dim,), jnp.float32),
            pltpu.VMEM((_CH * dim,), jnp.float32),
            pltpu.SemaphoreType.DMA,
            pltpu.SemaphoreType.DMA,
            pltpu.SemaphoreType.DMA,
            pltpu.SemaphoreType.DMA,
        ],
    )
    def k(tab_hbm, tail_hbm, rm_hbm, t0, t1, o0, o1, gs0, gs1, ss0, ss1):
        wid = lax.axis_index("s") * _NC + lax.axis_index("c")
        lanes = lax.iota(jnp.int32, 16)
        tb = (t0, t1)
        ob = (o0, o1)
        gsem = (gs0, gs1)
        ssem = (ss0, ss1)

        def start_gather(t, buf):
            @pl.when(t < nchunks)
            def _():
                off = pl.multiple_of(t * _CH, _CH)
                pltpu.async_copy(
                    tab_hbm.at[:, pl.ds(off, _CH)],
                    tb[buf].at[:, pl.ds(0, _CH)], gsem[buf])

        def do_chunk(t, buf, drain_prev):
            @pl.when(t < nchunks)
            def _():
                pltpu.make_async_copy(
                    tab_hbm.at[:, pl.ds(0, _CH)],
                    tb[buf].at[:, pl.ds(0, _CH)], gsem[buf]).wait()

                @pl.when(drain_prev)
                def _():
                    pltpu.make_async_copy(
                        ob[buf], rm_hbm.at[pl.ds(0, _CH * dim)],
                        ssem[buf]).wait()

                src = tb[buf]
                dst = ob[buf]

                @plsc.parallel_loop(0, _CH // 8, 1, unroll=2)
                def _tr(g):
                    i0 = g * 8
                    vals = []
                    for r in range(8):
                        colv = jnp.full((16,), i0 + r, jnp.int32)
                        vals.append(plsc.load_gather(src, [lanes, colv]))
                        vals.append(plsc.load_gather(src, [lanes + 16, colv]))
                    for r in range(8):
                        i = i0 + r
                        dst[pl.ds(i * dim, 16)] = vals[2 * r]
                        dst[pl.ds(i * dim + 16, 16)] = vals[2 * r + 1]

                roff = pl.multiple_of(t * _CH * dim, _CH * dim)
                pltpu.async_copy(
                    dst, rm_hbm.at[pl.ds(roff, _CH * dim)], ssem[buf])

        start_gather(wid, 0)

        @pl.loop(0, npairs)
        def _pair(p):
            t_a = wid + (2 * p) * _NW
            t_b = wid + (2 * p + 1) * _NW
            start_gather(t_b, 1)
            do_chunk(t_a, 0, p > 0)
            start_gather(t_b + _NW, 0)
            do_chunk(t_b, 1, p > 0)

        for buf in range(2):
            pltpu.make_async_copy(
                ob[buf], rm_hbm.at[pl.ds(0, _CH * dim)], ssem[buf]).wait()

        @pl.when(wid == 0)
        def _tail():
            pltpu.sync_copy(tail_hbm, o0.at[pl.ds(0, ntail)])
            pltpu.sync_copy(o0.at[pl.ds(0, ntail)],
                            rm_hbm.at[pl.ds(vmain * dim, ntail)])

    return k(tab_t, tail.reshape(-1))


@functools.partial(jax.jit, static_argnames=("hist", "batch", "dim"))
def _sc_embed(xt, rm, hist, batch, dim):
    nq = batch // _BW
    nblocks = hist * nq
    reps = -(-nblocks // _NW)
    mesh = plsc.VectorSubcoreMesh(core_axis_name="c", subcore_axis_name="s")

    @functools.partial(
        pl.kernel,
        mesh=mesh,
        compiler_params=pltpu.CompilerParams(
            use_tc_tiling_on_sc=False, needs_layout_passes=False),
        out_type=jax.ShapeDtypeStruct((hist, dim, batch), jnp.float32),
        scratch_types=[
            pltpu.VMEM((_BW,), jnp.int32),
            pltpu.VMEM((_BW,), jnp.int32),
            pltpu.VMEM((_BW, dim), jnp.float32),
            pltpu.VMEM((_BW, dim), jnp.float32),
            pltpu.VMEM((dim, _BW + 1), jnp.float32),
            pltpu.SemaphoreType.DMA,
            pltpu.SemaphoreType.DMA,
        ],
    )
    def k(rm_hbm, xt_hbm, out_hbm, i0v, i1v, g0, g1, ob, gs0, gs1):
        wid = lax.axis_index("s") * _NC + lax.axis_index("c")
        lanes = lax.iota(jnp.int32, 16)
        iv = (i0v, i1v)
        gb = (g0, g1)
        gsem = (gs0, gs1)
        gather = [None, None]

        def block_start(rep, buf):
            t = wid + rep * _NW

            @pl.when(t < nblocks)
            def _():
                h = t // nq
                b0 = pl.multiple_of((t % nq) * _BW, _BW)
                pltpu.sync_copy(xt_hbm.at[h, pl.ds(b0, _BW)], iv[buf])
                gather[buf] = pltpu.async_copy(
                    rm_hbm.at[iv[buf]], gb[buf], gsem[buf])

        block_start(0, 0)
        for rep in range(reps):
            cur = rep % 2
            nxt = (rep + 1) % 2
            t = wid + rep * _NW
            if rep + 1 < reps:
                block_start(rep + 1, nxt)

            @pl.when(t < nblocks)
            def _work():
                h = t // nq
                b0 = pl.multiple_of((t % nq) * _BW, _BW)
                gather[cur].wait()
                src = gb[cur]
                lo16 = lanes
                hi16 = lanes + 16

                @plsc.parallel_loop(0, _BW // 8, 1, unroll=2)
                def _tr(g):
                    i0 = g * 8
                    for r in range(8):
                        i = i0 + r
                        colv = jnp.full((16,), i, jnp.int32)
                        lo = src[i, pl.ds(0, 16)]
                        hi = src[i, pl.ds(16, 16)]
                        plsc.store_scatter(ob, [lo16, colv], lo)
                        plsc.store_scatter(ob, [hi16, colv], hi)

                pltpu.sync_copy(ob.at[:, pl.ds(0, _BW)],
                                out_hbm.at[h, :, pl.ds(b0, _BW)])

    return k(rm, xt)


def kernel(x, table):
    batch, hist = x.shape
    vocab, dim = table.shape
    xt = x.T.astype(jnp.int32)
    tab_t = table.T
    vmain = (vocab // 128) * 128
    tail = table[vmain:]
    rm = _sc_detile(tab_t, tail, vocab, dim).reshape(vocab, dim)
    out_phys = _sc_embed(xt, rm, hist, batch, dim)
    return out_phys.transpose(2, 0, 1)


# speedup vs baseline: 2.2225x; 2.2225x over previous
import functools

import jax
import jax.numpy as jnp
from jax import lax
from jax.experimental import pallas as pl
from jax.experimental.pallas import tpu as pltpu
from jax.experimental.pallas import tpu_sc as plsc

_INFO = plsc.get_sparse_core_info()
_NC = _INFO.num_cores
_NS = _INFO.num_subcores
_NW = _NC * _NS
_BW = 1024
_CH = 768


@functools.partial(jax.jit, static_argnames=("vocab", "dim"))
def _sc_detile(tab_t, tail, vocab, dim):
    vmain = (vocab // 128) * 128
    nchunks = vmain // _CH
    npairs = -(-nchunks // (2 * _NW))
    ntail = (vocab - vmain) * dim
    mesh = plsc.VectorSubcoreMesh(core_axis_name="c", subcore_axis_name="s")

    @functools.partial(
        pl.kernel,
        mesh=mesh,
        compiler_params=pltpu.CompilerParams(needs_layout_passes=False),
        out_type=jax.ShapeDtypeStruct((vocab * dim,), jnp.float32),
        scratch_types=[
            pltpu.VMEM((dim, _CH), jnp.float32),
            pltpu.VMEM((dim, _CH), jnp.float32),
            pltpu.VMEM((_CH * dim,), jnp.float32),
            pltpu.VMEM((_CH * dim,), jnp.float32),
            pltpu.SemaphoreType.DMA,
            pltpu.SemaphoreType.DMA,
            pltpu.SemaphoreType.DMA,
            pltpu.SemaphoreType.DMA,
        ],
    )
    def k(tab_hbm, tail_hbm, rm_hbm, t0, t1, o0, o1, gs0, gs1, ss0, ss1):
        wid = lax.axis_index("s") * _NC + lax.axis_index("c")
        lanes = lax.iota(jnp.int32, 16)
        tb = (t0, t1)
        ob = (o0, o1)
        gsem = (gs0, gs1)
        ssem = (ss0, ss1)

        def start_gather(t, buf):
            @pl.when(t < nchunks)
            def _():
                off = pl.multiple_of(t * _CH, _CH)
                pltpu.async_copy(
                    tab_hbm.at[:, pl.ds(off, _CH)], tb[buf], gsem[buf])

        def do_chunk(t, buf, drain_prev):
            @pl.when(t < nchunks)
            def _():
                pltpu.make_async_copy(
                    tab_hbm.at[:, pl.ds(0, _CH)], tb[buf], gsem[buf]).wait()

                @pl.when(drain_prev)
                def _():
                    pltpu.make_async_copy(
                        ob[buf], rm_hbm.at[pl.ds(0, _CH * dim)],
                        ssem[buf]).wait()

                src = tb[buf]
                dst = ob[buf]

                @plsc.parallel_loop(0, _CH // 16, 1, unroll=2)
                def _tr(g):
                    pre = g * 16 + lanes
                    rowbase = pre * dim
                    for d in range(dim):
                        val = src[d, pl.ds(g * 16, 16)]
                        rot = (pre + d) & (dim - 1)
                        plsc.store_scatter(dst, [rowbase + rot], val)

                roff = pl.multiple_of(t * _CH * dim, _CH * dim)
                pltpu.async_copy(
                    dst, rm_hbm.at[pl.ds(roff, _CH * dim)], ssem[buf])

        start_gather(wid, 0)

        @pl.loop(0, npairs)
        def _pair(p):
            t_a = wid + (2 * p) * _NW
            t_b = wid + (2 * p + 1) * _NW
            start_gather(t_b, 1)
            do_chunk(t_a, 0, p > 0)
            start_gather(t_b + _NW, 0)
            do_chunk(t_b, 1, p > 0)

        for buf in range(2):
            pltpu.make_async_copy(
                ob[buf], rm_hbm.at[pl.ds(0, _CH * dim)], ssem[buf]).wait()

        @pl.when(wid == 0)
        def _tail():
            pltpu.sync_copy(tail_hbm, o0.at[pl.ds(0, ntail)])
            pltpu.sync_copy(o0.at[pl.ds(0, ntail)],
                            rm_hbm.at[pl.ds(vmain * dim, ntail)])

    return k(tab_t, tail.reshape(-1))


@functools.partial(jax.jit, static_argnames=("hist", "batch", "dim"))
def _sc_embed(xt, rm, hist, batch, dim):
    nq = batch // _BW
    nblocks = hist * nq
    reps = -(-nblocks // _NW)
    mesh = plsc.VectorSubcoreMesh(core_axis_name="c", subcore_axis_name="s")

    @functools.partial(
        pl.kernel,
        mesh=mesh,
        compiler_params=pltpu.CompilerParams(
            use_tc_tiling_on_sc=False, needs_layout_passes=False),
        out_type=jax.ShapeDtypeStruct((hist, dim, batch), jnp.float32),
        scratch_types=[
            pltpu.VMEM((_BW,), jnp.int32),
            pltpu.VMEM((_BW,), jnp.int32),
            pltpu.VMEM((_BW, dim), jnp.float32),
            pltpu.VMEM((_BW, dim), jnp.float32),
            pltpu.VMEM((dim, _BW + 1), jnp.float32),
            pltpu.SMEM((_BW,), jnp.int32),
            pltpu.SemaphoreType.DMA,
            pltpu.SemaphoreType.DMA,
        ],
    )
    def k(rm_hbm, xt_hbm, out_hbm, i0v, i1v, g0, g1, ob, sidx, gs0, gs1):
        wid = lax.axis_index("s") * _NC + lax.axis_index("c")
        lanes = lax.iota(jnp.int32, 16)
        iv = (i0v, i1v)
        gb = (g0, g1)
        gsem = (gs0, gs1)
        gather = [None, None]

        def block_start(rep, buf):
            t = wid + rep * _NW

            @pl.when(t < nblocks)
            def _():
                h = t // nq
                b0 = pl.multiple_of((t % nq) * _BW, _BW)
                pltpu.sync_copy(xt_hbm.at[h, pl.ds(b0, _BW)], iv[buf])
                gather[buf] = pltpu.async_copy(
                    rm_hbm.at[iv[buf]], gb[buf], gsem[buf])

        block_start(0, 0)
        for rep in range(reps):
            cur = rep % 2
            nxt = (rep + 1) % 2
            t = wid + rep * _NW
            if rep + 1 < reps:
                block_start(rep + 1, nxt)

            @pl.when(t < nblocks)
            def _work():
                h = t // nq
                b0 = pl.multiple_of((t % nq) * _BW, _BW)
                gather[cur].wait()
                src = gb[cur]
                myidx = iv[cur]

                @plsc.parallel_loop(0, _BW // 16, 1, unroll=2)
                def _tr(g):
                    i0 = g * 16
                    vv = myidx[pl.ds(i0, 16)]
                    for r in range(16):
                        i = i0 + r
                        v = vv[r]
                        colv = jnp.full((16,), i, jnp.int32)
                        dlo = (lanes - v) & (dim - 1)
                        dhi = (lanes + 16 - v) & (dim - 1)
                        lo = src[i, pl.ds(0, 16)]
                        hi = src[i, pl.ds(16, 16)]
                        plsc.store_scatter(ob, [dlo, colv], lo)
                        plsc.store_scatter(ob, [dhi, colv], hi)

                pltpu.sync_copy(ob.at[:, pl.ds(0, _BW)],
                                out_hbm.at[h, :, pl.ds(b0, _BW)])

    return k(rm, xt)


def kernel(x, table):
    batch, hist = x.shape
    vocab, dim = table.shape
    xt = x.T.astype(jnp.int32)
    tab_t = table.T
    vmain = (vocab // 128) * 128
    tail = table[vmain:]
    j = jnp.arange(vocab - vmain)[:, None]
    p = jnp.arange(dim)[None, :]
    tail_rot = jnp.take_along_axis(tail, (p - j) % dim, axis=1)
    rm = _sc_detile(tab_t, tail_rot, vocab, dim).reshape(vocab, dim)
    out_phys = _sc_embed(xt, rm, hist, batch, dim)
    return out_phys.transpose(2, 0, 1)
